# baseline (speedup 1.0000x reference)
import functools

import jax
import jax.numpy as jnp
from jax import lax
from jax.experimental import pallas as pl
from jax.experimental.pallas import tpu as pltpu
from jax.experimental.pallas import tpu_sc as plsc

E = 16
K = 2
C = 768
F = 3072
N = 2048
CAP = 512
DUMMY = E * CAP
BUF_ROWS = (E + 1) * CAP
RT = 1024
NT = N // RT
FT = 3072
NFT = F // FT
SUB = 128
NW = 32


def _router_body(x_ref, wg_ref, flat_ref, wrep_ref, cnt_ref, tot_ref):
    p = pl.program_id(0)
    r = pl.program_id(1)

    @pl.when(jnp.logical_and(p == 0, r == 0))
    def _():
        tot_ref[...] = jnp.zeros_like(tot_ref)

    xb = x_ref[...]
    wg = wg_ref[...]
    logits = lax.dot_general(wg, xb, (((1,), (1,)), ((), ())),
                             preferred_element_type=jnp.float32)
    row = lax.broadcasted_iota(jnp.int32, (E, RT), 0)
    m0 = jnp.max(logits, axis=0, keepdims=True)
    idx0 = jnp.min(jnp.where(logits == m0, row, E), axis=0, keepdims=True)
    l2 = jnp.where(row == idx0, -jnp.inf, logits)
    m1 = jnp.max(l2, axis=0, keepdims=True)
    idx1 = jnp.min(jnp.where(l2 == m1, row, E), axis=0, keepdims=True)
    p0 = 1.0 / (1.0 + jnp.exp(m1 - m0))

    idxk = jnp.where(p == 0, idx0, idx1)
    pk = jnp.where(p == 0, p0, 1.0 - p0)
    mk = (row == idxk).astype(jnp.float32)

    a_io = lax.broadcasted_iota(jnp.int32, (RT, RT), 0)
    b_io = lax.broadcasted_iota(jnp.int32, (RT, RT), 1)
    upper = (a_io < b_io).astype(jnp.float32)
    prefix = lax.dot_general(mk, upper, (((1,), (0,)), ((), ())),
                             preferred_element_type=jnp.float32)
    slot_t = tot_ref[...] + prefix
    slot = jnp.sum(slot_t * mk, axis=0, keepdims=True)
    valid = slot < float(CAP)
    flat = jnp.where(valid, idxk * CAP + slot.astype(jnp.int32), DUMMY)
    wk = jnp.where(valid, pk, 0.0)
    flat_ref[0, 0, 0:1, :] = flat
    ident = (a_io == b_io).astype(jnp.float32)
    wcol = lax.dot_general(ident, wk, (((1,), (1,)), ((), ())),
                           preferred_element_type=jnp.float32)
    wrep_ref[0, :, :] = jnp.broadcast_to(wcol, (RT, 16))
    tot_ref[...] = tot_ref[...] + jnp.sum(mk, axis=1, keepdims=True)
    cnt_ref[...] = jnp.minimum(tot_ref[...], float(CAP))


def _router(x_flat, w_g):
    return pl.pallas_call(
        _router_body,
        grid=(K, NT),
        in_specs=[
            pl.BlockSpec((RT, C), lambda p, r: (r, 0)),
            pl.BlockSpec((E, C), lambda p, r: (0, 0)),
        ],
        out_specs=[
            pl.BlockSpec((1, 1, 1, RT), lambda p, r: (p, r, 0, 0)),
            pl.BlockSpec((1, RT, 16), lambda p, r: (p, r, 0)),
            pl.BlockSpec((E, RT), lambda p, r: (0, 0)),
        ],
        out_shape=[
            jax.ShapeDtypeStruct((K, NT, 1, RT), jnp.int32),
            jax.ShapeDtypeStruct((K, N, 16), jnp.float32),
            jax.ShapeDtypeStruct((E, RT), jnp.float32),
        ],
        scratch_shapes=[pltpu.VMEM((E, RT), jnp.float32)],
    )(x_flat, w_g)


def _dispatch(x_flat, flat):
    mesh = plsc.VectorSubcoreMesh(core_axis_name="c", subcore_axis_name="s")
    ppw = (K * N) // NW

    @functools.partial(
        pl.kernel, mesh=mesh,
        out_type=jax.ShapeDtypeStruct((BUF_ROWS, C), jnp.float32),
        scratch_types=[
            pltpu.VMEM((ppw // 2,), jnp.int32),
            pltpu.VMEM((ppw // 2,), jnp.int32),
            pltpu.VMEM((ppw // 2, C), jnp.float32),
            pltpu.VMEM((ppw // 2, C), jnp.float32),
            pltpu.SemaphoreType.DMA,
            pltpu.SemaphoreType.DMA,
            pltpu.SemaphoreType.DMA,
        ],
    )
    def k(x_hbm, flat_hbm, buf_hbm, ia_v, ib_v, ra_v, rb_v, sema, semb, semw):
        hp = ppw // 2
        wid = lax.axis_index("c") * 16 + lax.axis_index("s")
        base = wid * ppw
        tok = (wid % (N // ppw)) * ppw
        cpia = pltpu.async_copy(flat_hbm.at[pl.ds(base, hp)], ia_v, sema)
        cpxa = pltpu.async_copy(x_hbm.at[pl.ds(tok, hp)], ra_v, sema)
        cpib = pltpu.async_copy(flat_hbm.at[pl.ds(base + hp, hp)], ib_v, semb)
        cpxb = pltpu.async_copy(x_hbm.at[pl.ds(tok + hp, hp)], rb_v, semb)
        cpia.wait()
        cpxa.wait()
        wa = pltpu.async_copy(ra_v, buf_hbm.at[ia_v], semw)
        cpib.wait()
        cpxb.wait()
        wb = pltpu.async_copy(rb_v, buf_hbm.at[ib_v], semw)
        wa.wait()
        wb.wait()

    return k(x_flat, flat)


def _mlp_body(cnt_ref, in_ref, w1_ref, w2_ref, out_ref):
    e = pl.program_id(0)
    f = pl.program_id(1)
    cnt = cnt_ref[jnp.minimum(e, E - 1)]
    w1 = w1_ref[0].astype(jnp.bfloat16)
    w2 = w2_ref[0].astype(jnp.bfloat16)
    for sub in range(CAP // SUB):
        @pl.when(jnp.logical_and(e < E, cnt > sub * SUB))
        def _():
            xin = in_ref[sub * SUB:(sub + 1) * SUB, :].astype(jnp.bfloat16)
            h = lax.dot(xin, w1, preferred_element_type=jnp.float32)
            h = jnp.maximum(h, 0.0)
            h = (h * h).astype(jnp.bfloat16)
            o = lax.dot(h, w2, preferred_element_type=jnp.float32)

            @pl.when(f == 0)
            def _():
                out_ref[sub * SUB:(sub + 1) * SUB, :] = o

            @pl.when(f > 0)
            def _():
                out_ref[sub * SUB:(sub + 1) * SUB, :] += o

    @pl.when(jnp.logical_and(e == E, f == 0))
    def _():
        out_ref[...] = jnp.zeros_like(out_ref)


def _mlp(counts, buf, c_fc, c_proj):
    grid_spec = pltpu.PrefetchScalarGridSpec(
        num_scalar_prefetch=1,
        grid=(E + 1, NFT),
        in_specs=[
            pl.BlockSpec((CAP, C), lambda e, f, s: (jnp.minimum(e, E - 1), 0)),
            pl.BlockSpec((1, C, FT), lambda e, f, s: (jnp.minimum(e, E - 1), 0, f)),
            pl.BlockSpec((1, FT, C), lambda e, f, s: (jnp.minimum(e, E - 1), f, 0)),
        ],
        out_specs=pl.BlockSpec((CAP, C), lambda e, f, s: (e, 0)),
    )
    return pl.pallas_call(
        _mlp_body,
        grid_spec=grid_spec,
        out_shape=jax.ShapeDtypeStruct((BUF_ROWS, C), jnp.float32),
    )(counts, buf, c_fc, c_proj)


def _combine(eo, flat, wrep):
    mesh = plsc.VectorSubcoreMesh(core_axis_name="c", subcore_axis_name="s")
    tpw = N // NW

    nb = 4
    hb = tpw // nb

    @functools.partial(
        pl.kernel, mesh=mesh,
        out_type=jax.ShapeDtypeStruct((N, C), jnp.float32),
        scratch_types=(
            [pltpu.VMEM((hb,), jnp.int32) for _ in range(2 * nb)] +
            [
                pltpu.VMEM((tpw, 16), jnp.float32),
                pltpu.VMEM((tpw, 16), jnp.float32),
            ] +
            [pltpu.VMEM((hb, C), jnp.float32) for _ in range(2 * nb)] +
            [pltpu.SemaphoreType.DMA for _ in range(nb)] +
            [pltpu.SemaphoreType.DMA]
        ),
    )
    def k(eo_hbm, flat_hbm, wrep_hbm, out_hbm, *refs):
        f_v = refs[:2 * nb]
        w0_v, w1_v = refs[2 * nb:2 * nb + 2]
        r_v = refs[2 * nb + 2:4 * nb + 2]
        sems = refs[4 * nb + 2:5 * nb + 2]
        semw = refs[5 * nb + 2]
        wid = lax.axis_index("c") * 16 + lax.axis_index("s")
        base = wid * tpw
        for b in range(nb):
            pltpu.sync_copy(flat_hbm.at[pl.ds(base + b * hb, hb)], f_v[b])
            pltpu.sync_copy(flat_hbm.at[pl.ds(N + base + b * hb, hb)],
                            f_v[nb + b])
        gathers = []
        for b in range(nb):
            gathers.append(pltpu.async_copy(eo_hbm.at[f_v[b]], r_v[b], sems[b]))
            gathers.append(pltpu.async_copy(eo_hbm.at[f_v[nb + b]],
                                            r_v[nb + b], sems[b]))
        pltpu.sync_copy(wrep_hbm.at[0, pl.ds(base, tpw)], w0_v)
        pltpu.sync_copy(wrep_hbm.at[1, pl.ds(base, tpw)], w1_v)

        def make_body(r0_ref, r1_ref, woff):
            def tok_body(i, _):
                w0 = w0_v[woff + i, :]
                w1 = w1_v[woff + i, :]
                for c2 in range(C // 16):
                    slc = pl.ds(c2 * 16, 16)
                    r0_ref[i, slc] = r0_ref[i, slc] * w0 + r1_ref[i, slc] * w1
                return 0
            return tok_body

        writes = []
        for b in range(nb):
            gathers[2 * b].wait()
            gathers[2 * b + 1].wait()
            lax.fori_loop(0, hb, make_body(r_v[b], r_v[nb + b], b * hb), 0)
            writes.append(pltpu.async_copy(
                r_v[b], out_hbm.at[pl.ds(base + b * hb, hb)], semw))
        for wr in writes:
            wr.wait()

    return k(eo, flat, wrep)


def kernel(x, w_g, c_fc, c_proj):
    x_flat = x.reshape(N, C)
    flat4, wrep, cntf = _router(x_flat, w_g)
    flat = flat4.reshape(K * N)
    counts = cntf[:, 0].astype(jnp.int32)
    buf = _dispatch(x_flat, flat)
    eo = _mlp(counts, buf, c_fc, c_proj)
    out = _combine(eo, flat, wrep)
    return out.reshape(1, N, C)


# speedup vs baseline: 1.0175x; 1.0175x over previous
import functools

import jax
import jax.numpy as jnp
from jax import lax
from jax.experimental import pallas as pl
from jax.experimental.pallas import tpu as pltpu
from jax.experimental.pallas import tpu_sc as plsc

E = 16
K = 2
C = 768
F = 3072
N = 2048
CAP = 512
DUMMY = E * CAP
BUF_ROWS = (E + 1) * CAP
RT = 1024
NT = N // RT
FT = 3072
NFT = F // FT
SUB = 128
NW = 32


def _router_body(x_ref, wg_ref, flat_ref, wrep_ref, cnt_ref, tot_ref):
    p = pl.program_id(0)
    r = pl.program_id(1)

    @pl.when(jnp.logical_and(p == 0, r == 0))
    def _():
        tot_ref[...] = jnp.zeros_like(tot_ref)

    xb = x_ref[...]
    wg = wg_ref[...]
    logits = lax.dot_general(wg, xb, (((1,), (1,)), ((), ())),
                             preferred_element_type=jnp.float32)
    row = lax.broadcasted_iota(jnp.int32, (E, RT), 0)
    m0 = jnp.max(logits, axis=0, keepdims=True)
    idx0 = jnp.min(jnp.where(logits == m0, row, E), axis=0, keepdims=True)
    l2 = jnp.where(row == idx0, -jnp.inf, logits)
    m1 = jnp.max(l2, axis=0, keepdims=True)
    idx1 = jnp.min(jnp.where(l2 == m1, row, E), axis=0, keepdims=True)
    p0 = 1.0 / (1.0 + jnp.exp(m1 - m0))

    idxk = jnp.where(p == 0, idx0, idx1)
    pk = jnp.where(p == 0, p0, 1.0 - p0)
    mk = (row == idxk).astype(jnp.float32)

    a_io = lax.broadcasted_iota(jnp.int32, (RT, RT), 0)
    b_io = lax.broadcasted_iota(jnp.int32, (RT, RT), 1)
    upper = (a_io < b_io).astype(jnp.float32)
    prefix = lax.dot_general(mk, upper, (((1,), (0,)), ((), ())),
                             preferred_element_type=jnp.float32)
    slot_t = tot_ref[...] + prefix
    slot = jnp.sum(slot_t * mk, axis=0, keepdims=True)
    valid = slot < float(CAP)
    flat = jnp.where(valid, idxk * CAP + slot.astype(jnp.int32), DUMMY)
    wk = jnp.where(valid, pk, 0.0)
    flat_ref[0, 0, 0:1, :] = flat
    ident = (a_io == b_io).astype(jnp.float32)
    wcol = lax.dot_general(ident, wk, (((1,), (1,)), ((), ())),
                           preferred_element_type=jnp.float32)
    wrep_ref[0, :, :] = jnp.broadcast_to(wcol, (RT, 16))
    tot_ref[...] = tot_ref[...] + jnp.sum(mk, axis=1, keepdims=True)
    cnt_ref[...] = jnp.minimum(tot_ref[...], float(CAP))


def _router(x_flat, w_g):
    return pl.pallas_call(
        _router_body,
        grid=(K, NT),
        in_specs=[
            pl.BlockSpec((RT, C), lambda p, r: (r, 0)),
            pl.BlockSpec((E, C), lambda p, r: (0, 0)),
        ],
        out_specs=[
            pl.BlockSpec((1, 1, 1, RT), lambda p, r: (p, r, 0, 0)),
            pl.BlockSpec((1, RT, 16), lambda p, r: (p, r, 0)),
            pl.BlockSpec((E, RT), lambda p, r: (0, 0)),
        ],
        out_shape=[
            jax.ShapeDtypeStruct((K, NT, 1, RT), jnp.int32),
            jax.ShapeDtypeStruct((K, N, 16), jnp.float32),
            jax.ShapeDtypeStruct((E, RT), jnp.float32),
        ],
        scratch_shapes=[pltpu.VMEM((E, RT), jnp.float32)],
    )(x_flat, w_g)


def _dispatch(x_flat, flat):
    mesh = plsc.VectorSubcoreMesh(core_axis_name="c", subcore_axis_name="s")
    ppw = (K * N) // NW

    @functools.partial(
        pl.kernel, mesh=mesh,
        out_type=jax.ShapeDtypeStruct((BUF_ROWS, C), jnp.float32),
        scratch_types=[
            pltpu.VMEM((ppw // 2,), jnp.int32),
            pltpu.VMEM((ppw // 2,), jnp.int32),
            pltpu.VMEM((ppw // 2, C), jnp.float32),
            pltpu.VMEM((ppw // 2, C), jnp.float32),
            pltpu.SemaphoreType.DMA,
            pltpu.SemaphoreType.DMA,
            pltpu.SemaphoreType.DMA,
        ],
    )
    def k(x_hbm, flat_hbm, buf_hbm, ia_v, ib_v, ra_v, rb_v, sema, semb, semw):
        hp = ppw // 2
        wid = lax.axis_index("c") * 16 + lax.axis_index("s")
        base = wid * ppw
        tok = (wid % (N // ppw)) * ppw
        cpia = pltpu.async_copy(flat_hbm.at[pl.ds(base, hp)], ia_v, sema)
        cpxa = pltpu.async_copy(x_hbm.at[pl.ds(tok, hp)], ra_v, sema)
        cpib = pltpu.async_copy(flat_hbm.at[pl.ds(base + hp, hp)], ib_v, semb)
        cpxb = pltpu.async_copy(x_hbm.at[pl.ds(tok + hp, hp)], rb_v, semb)
        cpia.wait()
        cpxa.wait()
        wa = pltpu.async_copy(ra_v, buf_hbm.at[ia_v], semw)
        cpib.wait()
        cpxb.wait()
        wb = pltpu.async_copy(rb_v, buf_hbm.at[ib_v], semw)
        wa.wait()
        wb.wait()

    return k(x_flat, flat)


def _mlp_body(cnt_ref, in_ref, w1_ref, w2_ref, out_ref):
    e = pl.program_id(0)
    f = pl.program_id(1)
    cnt = cnt_ref[jnp.minimum(e, E - 1)]
    w1 = w1_ref[0].astype(jnp.bfloat16)
    w2 = w2_ref[0].astype(jnp.bfloat16)
    for sub in range(CAP // SUB):
        @pl.when(jnp.logical_and(e < E, cnt > sub * SUB))
        def _():
            xin = in_ref[sub * SUB:(sub + 1) * SUB, :].astype(jnp.bfloat16)
            h = lax.dot(xin, w1, preferred_element_type=jnp.float32)
            h = jnp.maximum(h, 0.0)
            h = (h * h).astype(jnp.bfloat16)
            o = lax.dot(h, w2, preferred_element_type=jnp.float32)

            @pl.when(f == 0)
            def _():
                out_ref[sub * SUB:(sub + 1) * SUB, :] = o

            @pl.when(f > 0)
            def _():
                out_ref[sub * SUB:(sub + 1) * SUB, :] += o

    @pl.when(jnp.logical_and(e == E, f == 0))
    def _():
        out_ref[...] = jnp.zeros_like(out_ref)


def _mlp(counts, buf, c_fc, c_proj):
    grid_spec = pltpu.PrefetchScalarGridSpec(
        num_scalar_prefetch=1,
        grid=(E + 1, NFT),
        in_specs=[
            pl.BlockSpec((CAP, C), lambda e, f, s: (jnp.minimum(e, E - 1), 0)),
            pl.BlockSpec((1, C, FT), lambda e, f, s: (jnp.minimum(e, E - 1), 0, f)),
            pl.BlockSpec((1, FT, C), lambda e, f, s: (jnp.minimum(e, E - 1), f, 0)),
        ],
        out_specs=pl.BlockSpec((CAP, C), lambda e, f, s: (e, 0)),
    )
    return pl.pallas_call(
        _mlp_body,
        grid_spec=grid_spec,
        out_shape=jax.ShapeDtypeStruct((BUF_ROWS, C), jnp.float32),
    )(counts, buf, c_fc, c_proj)


def _combine(eo, flat, wrep):
    mesh = plsc.VectorSubcoreMesh(core_axis_name="c", subcore_axis_name="s")
    tpw = N // NW

    nb = 2
    hb = tpw // nb

    @functools.partial(
        pl.kernel, mesh=mesh,
        out_type=jax.ShapeDtypeStruct((N, C), jnp.float32),
        scratch_types=(
            [pltpu.VMEM((hb,), jnp.int32) for _ in range(2 * nb)] +
            [
                pltpu.VMEM((tpw, 16), jnp.float32),
                pltpu.VMEM((tpw, 16), jnp.float32),
            ] +
            [pltpu.VMEM((hb, C), jnp.float32) for _ in range(2 * nb)] +
            [pltpu.SemaphoreType.DMA for _ in range(nb)] +
            [pltpu.SemaphoreType.DMA]
        ),
    )
    def k(eo_hbm, flat_hbm, wrep_hbm, out_hbm, *refs):
        f_v = refs[:2 * nb]
        w0_v, w1_v = refs[2 * nb:2 * nb + 2]
        r_v = refs[2 * nb + 2:4 * nb + 2]
        sems = refs[4 * nb + 2:5 * nb + 2]
        semw = refs[5 * nb + 2]
        wid = lax.axis_index("c") * 16 + lax.axis_index("s")
        base = wid * tpw
        for b in range(nb):
            pltpu.sync_copy(flat_hbm.at[pl.ds(base + b * hb, hb)], f_v[b])
            pltpu.sync_copy(flat_hbm.at[pl.ds(N + base + b * hb, hb)],
                            f_v[nb + b])
        gathers = []
        for b in range(nb):
            gathers.append(pltpu.async_copy(eo_hbm.at[f_v[b]], r_v[b], sems[b]))
            gathers.append(pltpu.async_copy(eo_hbm.at[f_v[nb + b]],
                                            r_v[nb + b], sems[b]))
        pltpu.sync_copy(wrep_hbm.at[0, pl.ds(base, tpw)], w0_v)
        pltpu.sync_copy(wrep_hbm.at[1, pl.ds(base, tpw)], w1_v)

        def make_body(r0_ref, r1_ref, woff):
            def tok_body(i, _):
                w0 = w0_v[woff + i, :]
                w1 = w1_v[woff + i, :]
                for c2 in range(C // 16):
                    slc = pl.ds(c2 * 16, 16)
                    r0_ref[i, slc] = r0_ref[i, slc] * w0 + r1_ref[i, slc] * w1
                return 0
            return tok_body

        writes = []
        for b in range(nb):
            gathers[2 * b].wait()
            gathers[2 * b + 1].wait()
            lax.fori_loop(0, hb, make_body(r_v[b], r_v[nb + b], b * hb), 0)
            writes.append(pltpu.async_copy(
                r_v[b], out_hbm.at[pl.ds(base + b * hb, hb)], semw))
        for wr in writes:
            wr.wait()

    return k(eo, flat, wrep)


def kernel(x, w_g, c_fc, c_proj):
    x_flat = x.reshape(N, C)
    flat4, wrep, cntf = _router(x_flat, w_g)
    flat = flat4.reshape(K * N)
    counts = cntf[:, 0].astype(jnp.int32)
    buf = _dispatch(x_flat, flat)
    eo = _mlp(counts, buf, c_fc, c_proj)
    out = _combine(eo, flat, wrep)
    return out.reshape(1, N, C)


# speedup vs baseline: 1.0191x; 1.0016x over previous
import functools

import jax
import jax.numpy as jnp
from jax import lax
from jax.experimental import pallas as pl
from jax.experimental.pallas import tpu as pltpu
from jax.experimental.pallas import tpu_sc as plsc

E = 16
K = 2
C = 768
F = 3072
N = 2048
CAP = 512
DUMMY = E * CAP
BUF_ROWS = (E + 1) * CAP
RT = 2048
NT = N // RT
FT = 3072
NFT = F // FT
SUB = 128
NW = 32


def _router_body(x_ref, wg_ref, flat_ref, wrep_ref, cnt_ref, tot_ref):
    p = pl.program_id(0)
    r = pl.program_id(1)

    @pl.when(jnp.logical_and(p == 0, r == 0))
    def _():
        tot_ref[...] = jnp.zeros_like(tot_ref)

    xb = x_ref[...]
    wg = wg_ref[...]
    logits = lax.dot_general(wg, xb, (((1,), (1,)), ((), ())),
                             preferred_element_type=jnp.float32)
    row = lax.broadcasted_iota(jnp.int32, (E, RT), 0)
    m0 = jnp.max(logits, axis=0, keepdims=True)
    idx0 = jnp.min(jnp.where(logits == m0, row, E), axis=0, keepdims=True)
    l2 = jnp.where(row == idx0, -jnp.inf, logits)
    m1 = jnp.max(l2, axis=0, keepdims=True)
    idx1 = jnp.min(jnp.where(l2 == m1, row, E), axis=0, keepdims=True)
    p0 = 1.0 / (1.0 + jnp.exp(m1 - m0))

    idxk = jnp.where(p == 0, idx0, idx1)
    pk = jnp.where(p == 0, p0, 1.0 - p0)
    mk = (row == idxk).astype(jnp.float32)

    a_io = lax.broadcasted_iota(jnp.int32, (RT, RT), 0)
    b_io = lax.broadcasted_iota(jnp.int32, (RT, RT), 1)
    upper = (a_io < b_io).astype(jnp.float32)
    prefix = lax.dot_general(mk, upper, (((1,), (0,)), ((), ())),
                             preferred_element_type=jnp.float32)
    slot_t = tot_ref[...] + prefix
    slot = jnp.sum(slot_t * mk, axis=0, keepdims=True)
    valid = slot < float(CAP)
    flat = jnp.where(valid, idxk * CAP + slot.astype(jnp.int32), DUMMY)
    wk = jnp.where(valid, pk, 0.0)
    flat_ref[0, 0, 0:1, :] = flat
    ident = (a_io == b_io).astype(jnp.float32)
    wcol = lax.dot_general(ident, wk, (((1,), (1,)), ((), ())),
                           preferred_element_type=jnp.float32)
    wrep_ref[0, :, :] = jnp.broadcast_to(wcol, (RT, 16))
    tot_ref[...] = tot_ref[...] + jnp.sum(mk, axis=1, keepdims=True)
    cnt_ref[...] = jnp.minimum(tot_ref[...], float(CAP))


def _router(x_flat, w_g):
    return pl.pallas_call(
        _router_body,
        grid=(K, NT),
        in_specs=[
            pl.BlockSpec((RT, C), lambda p, r: (r, 0)),
            pl.BlockSpec((E, C), lambda p, r: (0, 0)),
        ],
        out_specs=[
            pl.BlockSpec((1, 1, 1, RT), lambda p, r: (p, r, 0, 0)),
            pl.BlockSpec((1, RT, 16), lambda p, r: (p, r, 0)),
            pl.BlockSpec((E, RT), lambda p, r: (0, 0)),
        ],
        out_shape=[
            jax.ShapeDtypeStruct((K, NT, 1, RT), jnp.int32),
            jax.ShapeDtypeStruct((K, N, 16), jnp.float32),
            jax.ShapeDtypeStruct((E, RT), jnp.float32),
        ],
        scratch_shapes=[pltpu.VMEM((E, RT), jnp.float32)],
    )(x_flat, w_g)


def _dispatch(x_flat, flat):
    mesh = plsc.VectorSubcoreMesh(core_axis_name="c", subcore_axis_name="s")
    ppw = (K * N) // NW

    @functools.partial(
        pl.kernel, mesh=mesh,
        out_type=jax.ShapeDtypeStruct((BUF_ROWS, C), jnp.float32),
        scratch_types=[
            pltpu.VMEM((ppw // 2,), jnp.int32),
            pltpu.VMEM((ppw // 2,), jnp.int32),
            pltpu.VMEM((ppw // 2, C), jnp.float32),
            pltpu.VMEM((ppw // 2, C), jnp.float32),
            pltpu.SemaphoreType.DMA,
            pltpu.SemaphoreType.DMA,
            pltpu.SemaphoreType.DMA,
        ],
    )
    def k(x_hbm, flat_hbm, buf_hbm, ia_v, ib_v, ra_v, rb_v, sema, semb, semw):
        hp = ppw // 2
        wid = lax.axis_index("c") * 16 + lax.axis_index("s")
        base = wid * ppw
        tok = (wid % (N // ppw)) * ppw
        cpia = pltpu.async_copy(flat_hbm.at[pl.ds(base, hp)], ia_v, sema)
        cpxa = pltpu.async_copy(x_hbm.at[pl.ds(tok, hp)], ra_v, sema)
        cpib = pltpu.async_copy(flat_hbm.at[pl.ds(base + hp, hp)], ib_v, semb)
        cpxb = pltpu.async_copy(x_hbm.at[pl.ds(tok + hp, hp)], rb_v, semb)
        cpia.wait()
        cpxa.wait()
        wa = pltpu.async_copy(ra_v, buf_hbm.at[ia_v], semw)
        cpib.wait()
        cpxb.wait()
        wb = pltpu.async_copy(rb_v, buf_hbm.at[ib_v], semw)
        wa.wait()
        wb.wait()

    return k(x_flat, flat)


def _mlp_body(cnt_ref, in_ref, w1_ref, w2_ref, out_ref):
    e = pl.program_id(0)
    f = pl.program_id(1)
    cnt = cnt_ref[jnp.minimum(e, E - 1)]
    w1 = w1_ref[0].astype(jnp.bfloat16)
    w2 = w2_ref[0].astype(jnp.bfloat16)
    for sub in range(CAP // SUB):
        @pl.when(jnp.logical_and(e < E, cnt > sub * SUB))
        def _():
            xin = in_ref[sub * SUB:(sub + 1) * SUB, :].astype(jnp.bfloat16)
            h = lax.dot(xin, w1, preferred_element_type=jnp.float32)
            h = jnp.maximum(h, 0.0)
            h = (h * h).astype(jnp.bfloat16)
            o = lax.dot(h, w2, preferred_element_type=jnp.float32)

            @pl.when(f == 0)
            def _():
                out_ref[sub * SUB:(sub + 1) * SUB, :] = o

            @pl.when(f > 0)
            def _():
                out_ref[sub * SUB:(sub + 1) * SUB, :] += o

    @pl.when(jnp.logical_and(e == E, f == 0))
    def _():
        out_ref[...] = jnp.zeros_like(out_ref)


def _mlp(counts, buf, c_fc, c_proj):
    grid_spec = pltpu.PrefetchScalarGridSpec(
        num_scalar_prefetch=1,
        grid=(E + 1, NFT),
        in_specs=[
            pl.BlockSpec((CAP, C), lambda e, f, s: (jnp.minimum(e, E - 1), 0)),
            pl.BlockSpec((1, C, FT), lambda e, f, s: (jnp.minimum(e, E - 1), 0, f)),
            pl.BlockSpec((1, FT, C), lambda e, f, s: (jnp.minimum(e, E - 1), f, 0)),
        ],
        out_specs=pl.BlockSpec((CAP, C), lambda e, f, s: (e, 0)),
    )
    return pl.pallas_call(
        _mlp_body,
        grid_spec=grid_spec,
        out_shape=jax.ShapeDtypeStruct((BUF_ROWS, C), jnp.float32),
    )(counts, buf, c_fc, c_proj)


def _combine(eo, flat, wrep):
    mesh = plsc.VectorSubcoreMesh(core_axis_name="c", subcore_axis_name="s")
    tpw = N // NW

    nb = 2
    hb = tpw // nb

    @functools.partial(
        pl.kernel, mesh=mesh,
        out_type=jax.ShapeDtypeStruct((N, C), jnp.float32),
        scratch_types=(
            [pltpu.VMEM((hb,), jnp.int32) for _ in range(2 * nb)] +
            [
                pltpu.VMEM((tpw, 16), jnp.float32),
                pltpu.VMEM((tpw, 16), jnp.float32),
            ] +
            [pltpu.VMEM((hb, C), jnp.float32) for _ in range(2 * nb)] +
            [pltpu.SemaphoreType.DMA for _ in range(nb)] +
            [pltpu.SemaphoreType.DMA]
        ),
    )
    def k(eo_hbm, flat_hbm, wrep_hbm, out_hbm, *refs):
        f_v = refs[:2 * nb]
        w0_v, w1_v = refs[2 * nb:2 * nb + 2]
        r_v = refs[2 * nb + 2:4 * nb + 2]
        sems = refs[4 * nb + 2:5 * nb + 2]
        semw = refs[5 * nb + 2]
        wid = lax.axis_index("c") * 16 + lax.axis_index("s")
        base = wid * tpw
        for b in range(nb):
            pltpu.sync_copy(flat_hbm.at[pl.ds(base + b * hb, hb)], f_v[b])
            pltpu.sync_copy(flat_hbm.at[pl.ds(N + base + b * hb, hb)],
                            f_v[nb + b])
        gathers = []
        for b in range(nb):
            gathers.append(pltpu.async_copy(eo_hbm.at[f_v[b]], r_v[b], sems[b]))
            gathers.append(pltpu.async_copy(eo_hbm.at[f_v[nb + b]],
                                            r_v[nb + b], sems[b]))
        pltpu.sync_copy(wrep_hbm.at[0, pl.ds(base, tpw)], w0_v)
        pltpu.sync_copy(wrep_hbm.at[1, pl.ds(base, tpw)], w1_v)

        def make_body(r0_ref, r1_ref, woff):
            def tok_body(i, _):
                w0 = w0_v[woff + i, :]
                w1 = w1_v[woff + i, :]
                for c2 in range(C // 16):
                    slc = pl.ds(c2 * 16, 16)
                    r0_ref[i, slc] = r0_ref[i, slc] * w0 + r1_ref[i, slc] * w1
                return 0
            return tok_body

        writes = []
        for b in range(nb):
            gathers[2 * b].wait()
            gathers[2 * b + 1].wait()
            lax.fori_loop(0, hb, make_body(r_v[b], r_v[nb + b], b * hb), 0)
            writes.append(pltpu.async_copy(
                r_v[b], out_hbm.at[pl.ds(base + b * hb, hb)], semw))
        for wr in writes:
            wr.wait()

    return k(eo, flat, wrep)


def kernel(x, w_g, c_fc, c_proj):
    x_flat = x.reshape(N, C)
    flat4, wrep, cntf = _router(x_flat, w_g)
    flat = flat4.reshape(K * N)
    counts = cntf[:, 0].astype(jnp.int32)
    buf = _dispatch(x_flat, flat)
    eo = _mlp(counts, buf, c_fc, c_proj)
    out = _combine(eo, flat, wrep)
    return out.reshape(1, N, C)
